# baseline (speedup 1.0000x reference)
import jax
import jax.numpy as jnp
from jax.experimental import pallas as pl
from jax.experimental.pallas import tpu as pltpu

_LOG2E_HALF = 0.7213475204444817
_NQ = 4


def _cim_kernel(x_hbm, w_hbm, o_hbm, x_v, w_v, o_v, x_sem, w_sem, o_sem):
    b = x_v.shape[0]
    n = w_v.shape[0]
    nq = n // _NQ

    x_copy = pltpu.make_async_copy(x_hbm, x_v, x_sem)
    x_copy.start()
    w_copies = []
    for q in range(_NQ):
        sl = pl.ds(q * nq, nq)
        cp = pltpu.make_async_copy(w_hbm.at[sl, :], w_v.at[sl, :], w_sem.at[q])
        cp.start()
        w_copies.append(cp)

    x_copy.wait()
    x = x_v[...]
    xb = x.astype(jnp.bfloat16)
    xn = jnp.sum(x * x, axis=1, keepdims=True)

    o_copies = []
    for q in range(_NQ):
        w_copies[q].wait()
        w = w_v[pl.ds(q * nq, nq), :]
        wn = jnp.sum(w * w, axis=1)[None, :]
        dot = jax.lax.dot_general(
            xb, w.astype(jnp.bfloat16),
            (((1,), (1,)), ((), ())),
            preferred_element_type=jnp.float32,
        )
        sq = jnp.maximum(xn + wn - 2.0 * dot, 0.0)
        t = (1.0 + 1e-8) - jnp.exp2(sq * -_LOG2E_HALF)
        csl = pl.ds(q * nq, nq)
        o_v[:, csl] = t * jax.lax.rsqrt(t)
        cp = pltpu.make_async_copy(o_v.at[:, csl], o_hbm.at[:, csl], o_sem.at[q])
        cp.start()
        o_copies.append(cp)

    for cp in o_copies:
        cp.wait()


def kernel(x, weights):
    b, d = x.shape
    r, c, _ = weights.shape
    n = r * c
    w2 = weights.reshape(n, d)
    out = pl.pallas_call(
        _cim_kernel,
        in_specs=[
            pl.BlockSpec(memory_space=pl.ANY),
            pl.BlockSpec(memory_space=pl.ANY),
        ],
        out_specs=pl.BlockSpec(memory_space=pl.ANY),
        out_shape=jax.ShapeDtypeStruct((b, n), jnp.float32),
        scratch_shapes=[
            pltpu.VMEM((b, d), jnp.float32),
            pltpu.VMEM((n, d), jnp.float32),
            pltpu.VMEM((b, n), jnp.float32),
            pltpu.SemaphoreType.DMA,
            pltpu.SemaphoreType.DMA((_NQ,)),
            pltpu.SemaphoreType.DMA((_NQ,)),
        ],
    )(x, w2)
    return out.reshape(b, r, c)


# speedup vs baseline: 1.0302x; 1.0302x over previous
import jax
import jax.numpy as jnp
from jax.experimental import pallas as pl
from jax.experimental.pallas import tpu as pltpu

_LOG2E_HALF = 0.7213475204444817


def _cim_kernel(x_ref, w_ref, o_ref, xb_ref, xn_ref):
    @pl.when(pl.program_id(0) == 0)
    def _():
        x = x_ref[...]
        xb_ref[...] = x.astype(jnp.bfloat16)
        xn_ref[...] = jnp.sum(x * x, axis=1, keepdims=True)

    w = w_ref[...]
    wn = jnp.sum(w * w, axis=1)[None, :]
    dot = jax.lax.dot_general(
        xb_ref[...], w.astype(jnp.bfloat16),
        (((1,), (1,)), ((), ())),
        preferred_element_type=jnp.float32,
    )
    sq = jnp.maximum(xn_ref[...] + wn - 2.0 * dot, 0.0)
    t = (1.0 + 1e-8) - jnp.exp2(sq * -_LOG2E_HALF)
    o_ref[...] = t * jax.lax.rsqrt(t)


def kernel(x, weights):
    b, d = x.shape
    r, c, _ = weights.shape
    n = r * c
    w2 = weights.reshape(n, d)
    nb = n // 2
    out = pl.pallas_call(
        _cim_kernel,
        grid=(2,),
        in_specs=[
            pl.BlockSpec((b, d), lambda i: (0, 0)),
            pl.BlockSpec((nb, d), lambda i: (i, 0)),
        ],
        out_specs=pl.BlockSpec((b, nb), lambda i: (0, i)),
        out_shape=jax.ShapeDtypeStruct((b, n), jnp.float32),
        scratch_shapes=[
            pltpu.VMEM((b, d), jnp.bfloat16),
            pltpu.VMEM((b, 1), jnp.float32),
        ],
    )(x, w2)
    return out.reshape(b, r, c)
